# baseline (speedup 1.0000x reference)
import jax
import jax.numpy as jnp
from jax import lax
from jax.experimental import pallas as pl
from jax.experimental.pallas import tpu as pltpu

_GRP = 128


def _gather_mm_body(idx_ref, table_ref, o_ref, emb_out_ref, emb_vmem, sem):
  n = emb_vmem.shape[0]
  dim = emb_vmem.shape[1]
  n_grp = n // _GRP

  def grp(g, c):
    base = g * _GRP
    for j in range(_GRP):
      r = idx_ref[base + j]
      pltpu.make_async_copy(
          table_ref.at[pl.ds(r, 1)],
          emb_vmem.at[pl.ds(base + j, 1)],
          sem,
      ).start()

    @pl.when(g > 0)
    def _drain_prev():
      pltpu.make_async_copy(
          table_ref.at[pl.ds(0, _GRP)],
          emb_vmem.at[pl.ds((g - 1) * _GRP, _GRP)],
          sem,
      ).wait()

    return c

  lax.fori_loop(0, n_grp, grp, 0, unroll=False)
  pltpu.make_async_copy(
      table_ref.at[pl.ds(0, _GRP)],
      emb_vmem.at[pl.ds((n_grp - 1) * _GRP, _GRP)],
      sem,
  ).wait()

  half = o_ref.shape[0]
  u = emb_vmem[0:half, :]
  it = emb_vmem[2 * half:4 * half, :]
  o_ref[...] = lax.dot_general(
      u, it,
      dimension_numbers=(((1,), (1,)), ((), ())),
      preferred_element_type=jnp.float32,
  )
  emb_out_ref[...] = lax.concatenate(
      [emb_vmem[...], jnp.zeros((n, 128 - dim), jnp.float32)], 1)


def _mm_body(emb_ref, it_ref, o_ref):
  o_ref[...] = lax.dot_general(
      emb_ref[...], it_ref[...],
      dimension_numbers=(((1,), (1,)), ((), ())),
      preferred_element_type=jnp.float32,
  )


@jax.jit
def kernel(id_embedding, user_tensor, item_tensor):
  batch = user_tensor.shape[0]
  half = batch // 2
  n = 2 * batch
  idx = jnp.concatenate(
      [user_tensor.astype(jnp.int32), item_tensor.astype(jnp.int32)])

  s0, emb_p = pl.pallas_call(
      _gather_mm_body,
      grid=(1,),
      in_specs=[
          pl.BlockSpec(memory_space=pltpu.SMEM),
          pl.BlockSpec(memory_space=pl.ANY),
      ],
      out_specs=[
          pl.BlockSpec((half, batch), lambda i: (0, 0)),
          pl.BlockSpec((n, 128), lambda i: (0, 0)),
      ],
      out_shape=[
          jax.ShapeDtypeStruct((half, batch), jnp.float32),
          jax.ShapeDtypeStruct((n, 128), jnp.float32),
      ],
      scratch_shapes=[
          pltpu.VMEM((n, 64), jnp.float32),
          pltpu.SemaphoreType.DMA,
      ],
  )(idx, id_embedding)

  s1 = pl.pallas_call(
      _mm_body,
      grid=(1,),
      in_specs=[
          pl.BlockSpec((half, 128), lambda i: (1, 0)),
          pl.BlockSpec((batch, 128), lambda i: (1, 0)),
      ],
      out_specs=pl.BlockSpec((half, batch), lambda i: (0, 0)),
      out_shape=jax.ShapeDtypeStruct((half, batch), jnp.float32),
  )(emb_p, emb_p)

  return jnp.concatenate([s0, s1], axis=0)
